# baseline (speedup 1.0000x reference)
import functools

import jax
import jax.numpy as jnp
import numpy as np
from jax import lax
from jax.experimental import pallas as pl
from jax.experimental.pallas import tpu as pltpu
from jax.experimental.pallas import tpu_sc as plsc

_K = 16
_N = 100
_SIGMA = 0.05
_B = 8
_D = 2048
_L = 16
_NCH = _D // _L
_SPT = _N // 4


def _draw_noise():
    noise = jax.random.normal(jax.random.key(42), (_B, _N, _D), dtype=jnp.float32)
    return noise * jnp.float32(_SIGMA)


def _scaled_noise_host():
    try:
        with jax.default_device(jax.devices("cpu")[0]):
            return np.asarray(_draw_noise())
    except Exception:
        pass
    try:
        return np.asarray(_draw_noise())
    except Exception:
        return None


_SN_CONST = _scaled_noise_host()


def _sc_body(x_hbm, sn_hbm, out_hbm, xrow, slab, sidx, tidx, res, cidx, shared):
    c = lax.axis_index("c")
    s = lax.axis_index("s")
    b = c * 4 + s // 4
    q = s % 4
    s0 = q * _SPT

    pltpu.sync_copy(x_hbm.at[b], xrow)
    pltpu.sync_copy(sn_hbm.at[b, pl.ds(s0, _SPT)], slab)

    lane = lax.iota(jnp.int32, _L)
    ones = jnp.full((_L,), 1.0, jnp.float32)
    neginf = jnp.full((_L,), -jnp.inf, jnp.float32)

    groups = [(1, 1), (2, 2), (4, 4), (8, 8)]
    groups += [(g, 16) for g in range(16, _NCH, 16)]

    fifteen = jnp.full((_L,), _L - 1, jnp.int32)

    def sample_body(r, _):
        v0 = slab[r, pl.ds(0, _L)] + xrow[pl.ds(0, _L)]
        tv, ti = plsc.sort_key_val(v0, lane, descending=True)
        t = jnp.take(tv, fifteen)

        rfull = jnp.full((_L,), 1, jnp.int32) * r

        def collect_n(cnt, kks):
            ms, ss, bases = [], [], []
            for kk in kks:
                base = kk * _L
                v = slab[r, pl.ds(base, _L)] + xrow[pl.ds(base, _L)]
                m = v > t
                bases.append(base)
                ms.append(m)
                ss.append(jnp.sum(m.astype(jnp.int32)))
            offs = [cnt]
            for u in range(len(kks) - 1):
                offs.append(offs[-1] + ss[u])
            for u, base in enumerate(bases):
                plsc.store_compressed(
                    cidx.at[pl.ds(offs[u], _L)], lane + base, mask=ms[u]
                )
            return offs[-1] + ss[-1]

        def bmerge(mi, carry):
            tv, ti, cnt = carry
            off = mi * _L
            ci = cidx[pl.ds(off, _L)]
            valid = (lane + off) < cnt
            ci = jnp.where(valid, ci, 0)
            cv = plsc.load_gather(slab, [rfull, ci]) + plsc.load_gather(
                xrow, [ci]
            )
            cv = jnp.where(valid, cv, neginf)
            cv, ci = plsc.sort_key_val(cv, ci, descending=True)
            cvr = lax.rev(cv, (0,))
            cir = lax.rev(ci, (0,))
            take = tv >= cvr
            nv = jnp.where(take, tv, cvr)
            ni = jnp.where(take, ti, cir)
            nv, ni = plsc.sort_key_val(nv, ni, descending=True)
            return nv, ni, cnt

        for g0, glen in groups:
            cnt = collect_n(jnp.int32(0), list(range(g0, g0 + glen)))
            nb = (cnt + (_L - 1)) >> 4
            tv, ti, _ = lax.fori_loop(0, nb, bmerge, (tv, ti, cnt))
            t = jnp.take(tv, fifteen)

        si, _ = plsc.sort_key_val(ti, tv, descending=False)
        sidx[r] = si
        return 0

    lax.fori_loop(0, _SPT, sample_body, 0)

    sp = (s // 4) * 4
    rowbase = q * 4
    pltpu.sync_copy(sidx, shared.at[s])
    plsc.subcore_barrier()

    zero = jnp.zeros((_L,), jnp.float32)

    def zbody(i, _):
        row = i // (_D // (8 * _L))
        c0 = (i % (_D // (8 * _L))) * (8 * _L)
        for u in range(8):
            res[row, pl.ds(c0 + u * _L, _L)] = zero
        return 0

    lax.fori_loop(0, 4 * _D // (8 * _L), zbody, 0)

    jvec = (lane - rowbase) & 3
    jmask = (lane >= rowbase) & (lane < rowbase + 4)
    ones = jnp.full((_L,), 1.0, jnp.float32)

    for pj in range(4):
        pltpu.sync_copy(shared.at[sp + pj], tidx)

        def scat(r, _):
            plsc.addupdate_scatter(res, [jvec, tidx[r]], ones, mask=jmask)
            return 0

        lax.fori_loop(0, _SPT, scat, 0)

    def scale(i, _):
        row = i // (_D // (8 * _L))
        c0 = (i % (_D // (8 * _L))) * (8 * _L)
        for u in range(8):
            col = c0 + u * _L
            res[row, pl.ds(col, _L)] = res[row, pl.ds(col, _L)] * jnp.float32(
                1.0 / _N
            )
        return 0

    lax.fori_loop(0, 4 * _D // (8 * _L), scale, 0)

    pltpu.sync_copy(res, out_hbm.at[b, pl.ds(rowbase, 4)])


@jax.jit
def _run(x, sn):
    mesh = plsc.VectorSubcoreMesh(core_axis_name="c", subcore_axis_name="s")
    return pl.kernel(
        _sc_body,
        out_type=jax.ShapeDtypeStruct((_B, _K, _D), jnp.float32),
        mesh=mesh,
        scratch_types=[
            pltpu.VMEM((_D,), jnp.float32),
            pltpu.VMEM((_SPT, _D), jnp.float32),
            pltpu.VMEM((_SPT, _L), jnp.int32),
            pltpu.VMEM((_SPT, _L), jnp.int32),
            pltpu.VMEM((4, _D), jnp.float32),
            pltpu.VMEM((272,), jnp.int32),
            pltpu.VMEM_SHARED((16, _SPT, _L), jnp.int32),
        ],
        compiler_params=pltpu.CompilerParams(
            use_tc_tiling_on_sc=False, needs_layout_passes=False
        ),
    )(x, sn)


def kernel(x):
    sn = jnp.asarray(_SN_CONST) if _SN_CONST is not None else _draw_noise()
    return _run(x, sn)


# speedup vs baseline: 1.7112x; 1.7112x over previous
import functools

import jax
import jax.numpy as jnp
import numpy as np
from jax import lax
from jax.experimental import pallas as pl
from jax.experimental.pallas import tpu as pltpu
from jax.experimental.pallas import tpu_sc as plsc

_K = 16
_N = 100
_SIGMA = 0.05
_B = 8
_D = 2048
_L = 16
_NCH = _D // _L
_SPT = _N // 4


def _draw_noise():
    noise = jax.random.normal(jax.random.key(42), (_B, _N, _D), dtype=jnp.float32)
    return noise * jnp.float32(_SIGMA)


def _scaled_noise_host():
    try:
        with jax.default_device(jax.devices("cpu")[0]):
            return np.asarray(_draw_noise())
    except Exception:
        pass
    try:
        return np.asarray(_draw_noise())
    except Exception:
        return None


_SN_CONST = _scaled_noise_host()


def _sc_body(x_hbm, sn_hbm, out_hbm, xrow, slab, sidx, tidx, res, cvals, cidx, shared):
    c = lax.axis_index("c")
    s = lax.axis_index("s")
    b = c * 4 + s // 4
    q = s % 4
    s0 = q * _SPT

    pltpu.sync_copy(x_hbm.at[b], xrow)
    pltpu.sync_copy(sn_hbm.at[b, pl.ds(s0, _SPT)], slab)

    lane = lax.iota(jnp.int32, _L)
    ones = jnp.full((_L,), 1.0, jnp.float32)
    neginf = jnp.full((_L,), -jnp.inf, jnp.float32)

    groups = [(1, 1), (2, 2), (4, 4), (8, 8)]
    groups += [(g, 16) for g in range(16, _NCH, 16)]

    fifteen = jnp.full((_L,), _L - 1, jnp.int32)

    def sample_body(r, _):
        v0 = slab[r, pl.ds(0, _L)] + xrow[pl.ds(0, _L)]
        tv, ti = plsc.sort_key_val(v0, lane, descending=True)
        t = jnp.take(tv, fifteen)

        def collect_n(cnt, kks):
            vs, ms, ss = [], [], []
            for kk in kks:
                base = kk * _L
                v = slab[r, pl.ds(base, _L)] + xrow[pl.ds(base, _L)]
                m = v > t
                vs.append((v, base))
                ms.append(m)
                ss.append(plsc.all_reduce_population_count(m)[0])
            offs = [cnt]
            for u in range(len(kks) - 1):
                offs.append(offs[-1] + ss[u])
            for u, (v, base) in enumerate(vs):
                plsc.store_compressed(cvals.at[pl.ds(offs[u], _L)], v, mask=ms[u])
                plsc.store_compressed(
                    cidx.at[pl.ds(offs[u], _L)], lane + base, mask=ms[u]
                )
            return offs[-1] + ss[-1]

        def bmerge(mi, carry):
            tv, ti, cnt = carry
            off = mi * _L
            cv = cvals[pl.ds(off, _L)]
            ci = cidx[pl.ds(off, _L)]
            valid = (lane + off) < cnt
            cv = jnp.where(valid, cv, neginf)
            cv, ci = plsc.sort_key_val(cv, ci, descending=True)
            cvr = lax.rev(cv, (0,))
            cir = lax.rev(ci, (0,))
            take = tv >= cvr
            nv = jnp.where(take, tv, cvr)
            ni = jnp.where(take, ti, cir)
            nv, ni = plsc.sort_key_val(nv, ni, descending=True)
            return nv, ni, cnt

        for g0, glen in groups:
            cnt = collect_n(jnp.int32(0), list(range(g0, g0 + glen)))
            nb = (cnt + (_L - 1)) // _L
            tv, ti, _ = lax.fori_loop(0, nb, bmerge, (tv, ti, cnt))
            t = jnp.take(tv, fifteen)

        si, _ = plsc.sort_key_val(ti, tv, descending=False)
        sidx[r] = si
        return 0

    lax.fori_loop(0, _SPT, sample_body, 0)

    sp = (s // 4) * 4
    rowbase = q * 4
    pltpu.sync_copy(sidx, shared.at[s])
    plsc.subcore_barrier()

    zero = jnp.zeros((_L,), jnp.float32)

    def zbody(i, _):
        row = i // (_D // (8 * _L))
        c0 = (i % (_D // (8 * _L))) * (8 * _L)
        for u in range(8):
            res[row, pl.ds(c0 + u * _L, _L)] = zero
        return 0

    lax.fori_loop(0, 4 * _D // (8 * _L), zbody, 0)

    jvec = (lane - rowbase) & 3
    jmask = (lane >= rowbase) & (lane < rowbase + 4)
    ones = jnp.full((_L,), 1.0, jnp.float32)

    for pj in range(4):
        pltpu.sync_copy(shared.at[sp + pj], tidx)

        def scat(r, _):
            plsc.addupdate_scatter(res, [jvec, tidx[r]], ones, mask=jmask)
            return 0

        lax.fori_loop(0, _SPT, scat, 0)

    def scale(i, _):
        row = i // (_D // (8 * _L))
        c0 = (i % (_D // (8 * _L))) * (8 * _L)
        for u in range(8):
            col = c0 + u * _L
            res[row, pl.ds(col, _L)] = res[row, pl.ds(col, _L)] * jnp.float32(
                1.0 / _N
            )
        return 0

    lax.fori_loop(0, 4 * _D // (8 * _L), scale, 0)

    pltpu.sync_copy(res, out_hbm.at[b, pl.ds(rowbase, 4)])


@jax.jit
def _run(x, sn):
    mesh = plsc.VectorSubcoreMesh(core_axis_name="c", subcore_axis_name="s")
    return pl.kernel(
        _sc_body,
        out_type=jax.ShapeDtypeStruct((_B, _K, _D), jnp.float32),
        mesh=mesh,
        scratch_types=[
            pltpu.VMEM((_D,), jnp.float32),
            pltpu.VMEM((_SPT, _D), jnp.float32),
            pltpu.VMEM((_SPT, _L), jnp.int32),
            pltpu.VMEM((_SPT, _L), jnp.int32),
            pltpu.VMEM((4, _D), jnp.float32),
            pltpu.VMEM((272,), jnp.float32),
            pltpu.VMEM((272,), jnp.int32),
            pltpu.VMEM_SHARED((16, _SPT, _L), jnp.int32),
        ],
        compiler_params=pltpu.CompilerParams(
            use_tc_tiling_on_sc=False, needs_layout_passes=False
        ),
    )(x, sn)


def kernel(x):
    sn = jnp.asarray(_SN_CONST) if _SN_CONST is not None else _draw_noise()
    return _run(x, sn)


# speedup vs baseline: 1.7248x; 1.0080x over previous
import functools

import jax
import jax.numpy as jnp
import numpy as np
from jax import lax
from jax.experimental import pallas as pl
from jax.experimental.pallas import tpu as pltpu
from jax.experimental.pallas import tpu_sc as plsc

_K = 16
_N = 100
_SIGMA = 0.05
_B = 8
_D = 2048
_L = 16
_NCH = _D // _L
_SPT = _N // 4


def _draw_noise():
    noise = jax.random.normal(jax.random.key(42), (_B, _N, _D), dtype=jnp.float32)
    return noise * jnp.float32(_SIGMA)


def _scaled_noise_host():
    try:
        with jax.default_device(jax.devices("cpu")[0]):
            return np.asarray(_draw_noise())
    except Exception:
        pass
    try:
        return np.asarray(_draw_noise())
    except Exception:
        return None


_SN_CONST = _scaled_noise_host()


def _sc_body(x_hbm, sn_hbm, out_hbm, xrow, slab, sidx, tidx, res, cvals, cidx, shared):
    c = lax.axis_index("c")
    s = lax.axis_index("s")
    b = c * 4 + s // 4
    q = s % 4
    s0 = q * _SPT

    pltpu.sync_copy(x_hbm.at[b], xrow)
    pltpu.sync_copy(sn_hbm.at[b, pl.ds(s0, _SPT)], slab)

    lane = lax.iota(jnp.int32, _L)
    ones = jnp.full((_L,), 1.0, jnp.float32)
    neginf = jnp.full((_L,), -jnp.inf, jnp.float32)

    groups = [(1, 1), (2, 2), (4, 4), (8, 8)]
    groups += [(g, 16) for g in range(16, _NCH, 16)]

    fifteen = jnp.full((_L,), _L - 1, jnp.int32)

    def sample_body(r, _):
        v0 = slab[r, pl.ds(0, _L)] + xrow[pl.ds(0, _L)]
        tv, ti = plsc.sort_key_val(v0, lane, descending=True)
        t = jnp.take(tv, fifteen)

        def collect_n(cnt, kks):
            vs, ms, ss = [], [], []
            for kk in kks:
                base = kk * _L
                v = slab[r, pl.ds(base, _L)] + xrow[pl.ds(base, _L)]
                m = v > t
                vs.append((v, base))
                ms.append(m)
                ss.append(plsc.all_reduce_population_count(m)[0])
            offs = [cnt]
            for u in range(len(kks) - 1):
                offs.append(offs[-1] + ss[u])
            for u, (v, base) in enumerate(vs):
                plsc.store_compressed(cvals.at[pl.ds(offs[u], _L)], v, mask=ms[u])
                plsc.store_compressed(
                    cidx.at[pl.ds(offs[u], _L)], lane + base, mask=ms[u]
                )
            return offs[-1] + ss[-1]

        def bmerge(mi, carry):
            tv, ti, cnt = carry
            off = mi * _L
            cv = cvals[pl.ds(off, _L)]
            ci = cidx[pl.ds(off, _L)]
            valid = (lane + off) < cnt
            cv = jnp.where(valid, cv, neginf)
            cv, ci = plsc.sort_key_val(cv, ci, descending=False)
            take = tv >= cv
            nv = jnp.where(take, tv, cv)
            ni = jnp.where(take, ti, ci)
            nv, ni = plsc.sort_key_val(nv, ni, descending=True)
            return nv, ni, cnt

        for g0, glen in groups:
            cnt = collect_n(jnp.int32(0), list(range(g0, g0 + glen)))
            nb = (cnt + (_L - 1)) // _L
            tv, ti, _ = lax.fori_loop(0, nb, bmerge, (tv, ti, cnt))
            t = jnp.take(tv, fifteen)

        si, _ = plsc.sort_key_val(ti, tv, descending=False)
        sidx[r] = si
        return 0

    lax.fori_loop(0, _SPT, sample_body, 0)

    sp = (s // 4) * 4
    rowbase = q * 4
    pltpu.sync_copy(sidx, shared.at[s])
    plsc.subcore_barrier()

    zero = jnp.zeros((_L,), jnp.float32)

    def zbody(i, _):
        row = i // (_D // (8 * _L))
        c0 = (i % (_D // (8 * _L))) * (8 * _L)
        for u in range(8):
            res[row, pl.ds(c0 + u * _L, _L)] = zero
        return 0

    lax.fori_loop(0, 4 * _D // (8 * _L), zbody, 0)

    jvec = (lane - rowbase) & 3
    jmask = (lane >= rowbase) & (lane < rowbase + 4)
    ones = jnp.full((_L,), 1.0, jnp.float32)

    for pj in range(4):
        pltpu.sync_copy(shared.at[sp + pj], tidx)

        def scat(r, _):
            plsc.addupdate_scatter(res, [jvec, tidx[r]], ones, mask=jmask)
            return 0

        lax.fori_loop(0, _SPT, scat, 0)

    def scale(i, _):
        row = i // (_D // (8 * _L))
        c0 = (i % (_D // (8 * _L))) * (8 * _L)
        for u in range(8):
            col = c0 + u * _L
            res[row, pl.ds(col, _L)] = res[row, pl.ds(col, _L)] * jnp.float32(
                1.0 / _N
            )
        return 0

    lax.fori_loop(0, 4 * _D // (8 * _L), scale, 0)

    pltpu.sync_copy(res, out_hbm.at[b, pl.ds(rowbase, 4)])


@jax.jit
def _run(x, sn):
    mesh = plsc.VectorSubcoreMesh(core_axis_name="c", subcore_axis_name="s")
    return pl.kernel(
        _sc_body,
        out_type=jax.ShapeDtypeStruct((_B, _K, _D), jnp.float32),
        mesh=mesh,
        scratch_types=[
            pltpu.VMEM((_D,), jnp.float32),
            pltpu.VMEM((_SPT, _D), jnp.float32),
            pltpu.VMEM((_SPT, _L), jnp.int32),
            pltpu.VMEM((_SPT, _L), jnp.int32),
            pltpu.VMEM((4, _D), jnp.float32),
            pltpu.VMEM((272,), jnp.float32),
            pltpu.VMEM((272,), jnp.int32),
            pltpu.VMEM_SHARED((16, _SPT, _L), jnp.int32),
        ],
        compiler_params=pltpu.CompilerParams(
            use_tc_tiling_on_sc=False, needs_layout_passes=False
        ),
    )(x, sn)


def kernel(x):
    sn = jnp.asarray(_SN_CONST) if _SN_CONST is not None else _draw_noise()
    return _run(x, sn)
